# baseline (speedup 1.0000x reference)
import functools

import jax
import jax.numpy as jnp
from jax import lax
from jax.experimental import pallas as pl
from jax.experimental.pallas import tpu as pltpu
from jax.experimental.pallas import tpu_sc as plsc

B = 16384
D = 64
NC = 2
NS = 16
NW = NC * NS
B_PER_W = B // NW
CHUNK = 128
NCHUNK = B_PER_W // CHUNK


def _sc_kernel(ph_hbm, pt_hbm, rl_hbm, nh_hbm, nt_hbm, yr_hbm,
               ent_hbm, rel_hbm, time_hbm, out_hbm,
               idx_v, rows_v, pos_v, neg_v, sem):
    wid = lax.axis_index("s") * NC + lax.axis_index("c")
    base = wid * B_PER_W

    idx_srcs = (ph_hbm, pt_hbm, rl_hbm, nh_hbm, nt_hbm, yr_hbm)
    tables = (ent_hbm, ent_hbm, rel_hbm, ent_hbm, ent_hbm, time_hbm)

    for c in range(NCHUNK):
        off = base + c * CHUNK
        for j in range(6):
            pltpu.sync_copy(idx_srcs[j].at[pl.ds(off, CHUNK)], idx_v.at[j])
        cps = [pltpu.async_copy(tables[j].at[idx_v.at[j]], rows_v.at[j], sem)
               for j in range(6)]
        for cp in cps:
            cp.wait()

        lanes = lax.iota(jnp.int32, 16)
        perms = [lanes ^ s for s in (1, 2, 4, 8)]

        def lane_sum(v):
            for p in perms:
                v = v + v.at[p].get(mode="promise_in_bounds")
            return v

        def body(e, _):
            ip_p = jnp.zeros((16,), jnp.float32)
            ip_n = jnp.zeros((16,), jnp.float32)
            dps, dns, ts = [], [], []
            for k in range(4):
                sl = pl.ds(k * 16, 16)
                h = rows_v[0, e, sl]
                tl = rows_v[1, e, sl]
                r = rows_v[2, e, sl]
                nh = rows_v[3, e, sl]
                nt = rows_v[4, e, sl]
                t = rows_v[5, e, sl]
                dp = h + r - tl
                dn = nh + r - nt
                ip_p = ip_p + dp * t
                ip_n = ip_n + dn * t
                dps.append(dp)
                dns.append(dn)
                ts.append(t)
            sp = lane_sum(ip_p)
            sn = lane_sum(ip_n)
            ap = jnp.zeros((16,), jnp.float32)
            an = jnp.zeros((16,), jnp.float32)
            for k in range(4):
                ap = ap + jnp.abs(dps[k] - ts[k] * sp)
                an = an + jnp.abs(dns[k] - ts[k] * sn)
            lane0 = lanes == 0
            eidx = jnp.full((16,), e, jnp.int32)
            plsc.store_scatter(pos_v, [eidx], lane_sum(ap), mask=lane0)
            plsc.store_scatter(neg_v, [eidx], lane_sum(an), mask=lane0)
            return 0

        lax.fori_loop(0, CHUNK, body, 0)
        pltpu.sync_copy(pos_v, out_hbm.at[0, pl.ds(off, CHUNK)])
        pltpu.sync_copy(neg_v, out_hbm.at[1, pl.ds(off, CHUNK)])


@jax.jit
def _run(ph, pt, rl, nh, nt, yr, ent, rel, time):
    mesh = plsc.VectorSubcoreMesh(core_axis_name="c", subcore_axis_name="s")
    kfn = functools.partial(
        pl.kernel,
        mesh=mesh,
        compiler_params=pltpu.CompilerParams(
            needs_layout_passes=False, use_tc_tiling_on_sc=False),
        out_type=jax.ShapeDtypeStruct((2, B), jnp.float32),
        scratch_types=[
            pltpu.VMEM((6, CHUNK), jnp.int32),
            pltpu.VMEM((6, CHUNK, D), jnp.float32),
            pltpu.VMEM((CHUNK,), jnp.float32),
            pltpu.VMEM((CHUNK,), jnp.float32),
            pltpu.SemaphoreType.DMA,
        ],
    )(_sc_kernel)
    return kfn(ph, pt, rl, nh, nt, yr, ent, rel, time)


def kernel(pos_head, pos_tail, rel, neg_head, neg_tail, start_year,
           ent_embeddings, rel_embeddings, time_embeddings):
    ph = pos_head.reshape(B)
    pt = pos_tail.reshape(B)
    rl = rel.reshape(B)
    nh = neg_head.reshape(B)
    nt = neg_tail.reshape(B)
    return _run(ph, pt, rl, nh, nt, start_year,
                ent_embeddings, rel_embeddings, time_embeddings)
